# baseline (speedup 1.0000x reference)
import functools

import jax
import jax.numpy as jnp
from jax import lax
from jax.experimental import pallas as pl
from jax.experimental.pallas import tpu as pltpu
from jax.experimental.pallas import tpu_sc as plsc

_NUM_CLASSES = 21
_THRESHOLD = 0.5
_NEGPOS_RATIO = 3
_V0 = 0.1
_V1 = 0.2
_B = 32
_P = 8732
_O = 12
_PPAD = 8736
_CH = _PPAD // 16
_U = 14
_CHU = _CH // _U

_INTERPRET = False


def _tc_body(loc_ref, conf_ref, paug_ref, targets_ref, targets_t_ref,
             ll_ref, lc_ref, ce_ref):
    b = pl.program_id(0)

    x1p = paug_ref[0:1, :]
    y1p = paug_ref[1:2, :]
    x2p = paug_ref[2:3, :]
    y2p = paug_ref[3:4, :]
    area_p = paug_ref[4:5, :]
    px = paug_ref[5, :]
    py = paug_ref[6, :]
    inv_v0pw = paug_ref[7, :]
    inv_v0ph = paug_ref[8, :]
    inv_pw = paug_ref[9, :]
    inv_ph = paug_ref[10, :]

    t = targets_ref[0]
    tx1 = t[:, 0:1]
    ty1 = t[:, 1:2]
    tx2 = t[:, 2:3]
    ty2 = t[:, 3:4]
    area_t = (tx2 - tx1) * (ty2 - ty1)
    iw = jnp.maximum(jnp.minimum(x2p, tx2) - jnp.maximum(x1p, tx1), 0.0)
    ih = jnp.maximum(jnp.minimum(y2p, ty2) - jnp.maximum(y1p, ty1), 0.0)
    inter = iw * ih
    iou = inter / (area_p + area_t - inter)

    oio = lax.broadcasted_iota(jnp.int32, (_O, _P), 0)
    lio = lax.broadcasted_iota(jnp.int32, (_O, _P), 1)

    bv0 = jnp.max(iou, axis=0)
    bti0 = jnp.min(jnp.where(iou == bv0[None, :], oio, _O), axis=0)
    m = jnp.max(iou, axis=1, keepdims=True)
    bpi = jnp.min(jnp.where(iou == m, lio, _P), axis=1, keepdims=True)

    fm = lio == bpi
    f_o = jnp.max(jnp.where(fm, oio, -1), axis=0)
    f_any = f_o >= 0
    bv = jnp.where(f_any, 2.0, bv0)
    bti = jnp.where(f_any, f_o, bti0)

    onehotf = (oio == bti[None, :]).astype(jnp.float32)
    matched = lax.dot_general(
        targets_t_ref[0], onehotf, (((1,), (0,)), ((), ())),
        preferred_element_type=jnp.float32)
    mx1 = matched[0, :]
    my1 = matched[1, :]
    mx2 = matched[2, :]
    my2 = matched[3, :]
    labf = matched[4, :]

    conf_t = jnp.where(bv < _THRESHOLD, 0.0, labf + 1.0)
    pos = conf_t > 0.0
    posf = pos.astype(jnp.float32)

    gcx = ((mx1 + mx2) * 0.5 - px) * inv_v0pw
    gcy = ((my1 + my2) * 0.5 - py) * inv_v0ph
    gw = jnp.log((mx2 - mx1) * inv_pw) * (1.0 / _V1)
    gh = jnp.log((my2 - my1) * inv_ph) * (1.0 / _V1)
    ll = jnp.float32(0.0)
    for c, g in enumerate((gcx, gcy, gw, gh)):
        d = loc_ref[0, c, :] - g
        ad = jnp.abs(d)
        sl1 = jnp.where(ad < 1.0, 0.5 * d * d, ad - 0.5)
        ll = ll + jnp.sum(sl1 * posf)

    conf = conf_ref[0]
    rmax = jnp.max(conf, axis=0)
    s = jnp.sum(jnp.exp(conf - rmax[None, :]), axis=0)
    lse = jnp.log(s) + rmax
    cio = lax.broadcasted_iota(jnp.int32, (_NUM_CLASSES, _P), 0)
    conf_t_i = conf_t.astype(jnp.int32)
    g = jnp.sum(jnp.where(cio == conf_t_i[None, :], conf, 0.0), axis=0)
    ce = lse - g

    ce_m = jnp.where(pos, -2.0, ce)
    ce_ref[0, 0, :] = jnp.concatenate(
        [ce_m, jnp.full((_PPAD - _P,), -3.0, jnp.float32)])

    @pl.when(b == 0)
    def _init():
        ll_ref[0, 0] = 0.0
        lc_ref[0, 0] = 0.0

    ll_ref[0, 0] += ll
    lc_ref[0, 0] += jnp.sum(ce * posf)


def _sc_mine(ce_hbm, out_hbm, ce_v, row_v, red_v):
    img = lax.axis_index("s") * 2 + lax.axis_index("c")
    pltpu.sync_copy(ce_hbm.at[img], ce_v)

    zero16 = jnp.zeros((16,), jnp.float32)
    li = lax.iota(jnp.int32, 16)
    bfly = [li ^ 8, li ^ 4, li ^ 2, li ^ 1]

    def lane_red(v, op):
        for idx in bfly:
            red_v[...] = v
            v = op(v, plsc.load_gather(red_v, [idx]))
        return v

    def scan0(i, carry):
        mx, npv = carry
        for j in range(_U):
            ch = ce_v[pl.ds(i * (16 * _U) + 16 * j, 16)]
            mx = jnp.maximum(mx, ch)
            npv = npv + jnp.where(ch == -2.0, 1.0, 0.0)
        return mx, npv

    mx16, np16 = lax.fori_loop(
        0, _CHU, scan0, (jnp.full((16,), -10.0, jnp.float32), zero16))
    npos = lane_red(np16, jnp.add)
    k = jnp.minimum(_NEGPOS_RATIO * npos, jnp.float32(_P - 1))
    k = jnp.minimum(k, jnp.float32(_P) - npos)

    def bis(_, carry):
        lo, hi = carry
        mid = 0.5 * (lo + hi)

        def cnt_body(i, acc):
            for j in range(_U):
                ch = ce_v[pl.ds(i * (16 * _U) + 16 * j, 16)]
                acc = acc + jnp.where(ch > mid, 1.0, 0.0)
            return acc

        cn = lane_red(lax.fori_loop(0, _CHU, cnt_body, zero16), jnp.add)
        geq = cn >= k
        return jnp.where(geq, mid, lo), jnp.where(geq, hi, mid)

    lo, hi = lax.fori_loop(
        0, 24, bis,
        (jnp.full((16,), -1.0, jnp.float32),
         lane_red(mx16, jnp.maximum) + 1.0))

    def fin(i, carry):
        cnt, ssum = carry
        for j in range(_U):
            ch = ce_v[pl.ds(i * (16 * _U) + 16 * j, 16)]
            gt = ch > hi
            cnt = cnt + jnp.where(gt, 1.0, 0.0)
            ssum = ssum + jnp.where(gt, ch, 0.0)
        return cnt, ssum

    cnt16, ssum16 = lax.fori_loop(0, _CHU, fin, (zero16, zero16))
    s_top = lane_red(ssum16, jnp.add) + (k - lane_red(cnt16, jnp.add)) * hi

    row_v[...] = jnp.where(li == 0, s_top, jnp.where(li == 1, npos, 0.0))
    pltpu.sync_copy(row_v, out_hbm.at[img])


@jax.jit
def kernel(loc_data, conf_data, priors, targets):
    loc_t = jnp.transpose(loc_data, (0, 2, 1))
    conf_t2 = jnp.transpose(conf_data, (0, 2, 1))
    targets_t = jnp.transpose(targets, (0, 2, 1))

    px, py, pw, ph = priors[:, 0], priors[:, 1], priors[:, 2], priors[:, 3]
    paug = jnp.stack([
        px - pw * 0.5, py - ph * 0.5, px + pw * 0.5, py + ph * 0.5,
        pw * ph, px, py,
        1.0 / (_V0 * pw), 1.0 / (_V0 * ph), 1.0 / pw, 1.0 / ph,
    ], axis=0)

    scalar_spec = pl.BlockSpec((1, 1), lambda b: (0, 0),
                               memory_space=pltpu.SMEM)
    ll, lc_pos, ce_all = pl.pallas_call(
        _tc_body,
        grid=(_B,),
        in_specs=[
            pl.BlockSpec((1, 4, _P), lambda b: (b, 0, 0)),
            pl.BlockSpec((1, _NUM_CLASSES, _P), lambda b: (b, 0, 0)),
            pl.BlockSpec((11, _P), lambda b: (0, 0)),
            pl.BlockSpec((1, _O, 5), lambda b: (b, 0, 0)),
            pl.BlockSpec((1, 5, _O), lambda b: (b, 0, 0)),
        ],
        out_specs=[scalar_spec, scalar_spec,
                   pl.BlockSpec((1, 1, _PPAD), lambda b: (b, 0, 0))],
        out_shape=[jax.ShapeDtypeStruct((1, 1), jnp.float32),
                   jax.ShapeDtypeStruct((1, 1), jnp.float32),
                   jax.ShapeDtypeStruct((_B, 1, _PPAD), jnp.float32)],
        interpret=_INTERPRET,
    )(loc_t, conf_t2, paug, targets, targets_t)

    mine = pl.kernel(
        _sc_mine,
        out_type=jax.ShapeDtypeStruct((_B, 16), jnp.float32),
        mesh=plsc.VectorSubcoreMesh(core_axis_name="c",
                                    subcore_axis_name="s"),
        scratch_types=[pltpu.VMEM((_PPAD,), jnp.float32),
                       pltpu.VMEM((16,), jnp.float32),
                       pltpu.VMEM((16,), jnp.float32)],
        compiler_params=pltpu.CompilerParams(needs_layout_passes=False),
    )
    rows = mine(jnp.reshape(ce_all, (_B, _PPAD)))

    n = jnp.sum(rows[:, 1])
    lc = lc_pos[0, 0] + jnp.sum(rows[:, 0])
    return (ll[0, 0] / n, lc / n)


# speedup vs baseline: 1.0017x; 1.0017x over previous
import jax
import jax.numpy as jnp
from jax import lax
from jax.experimental import pallas as pl
from jax.experimental.pallas import tpu as pltpu
from jax.experimental.pallas import tpu_sc as plsc

_NUM_CLASSES = 21
_THRESHOLD = 0.5
_NEGPOS_RATIO = 3
_V0 = 0.1
_V1 = 0.2
_B = 32
_P = 8732
_O = 12
_PPAD = 8736
_CH = _PPAD // 16
_U = 7
_CHU = _CH // _U


def _tc_body(loc_ref, conf_ref, paug_ref, targets_ref, targets_t_ref,
             ll_ref, lc_ref, ce_ref):
    b = pl.program_id(0)

    x1p = paug_ref[0:1, :]
    y1p = paug_ref[1:2, :]
    x2p = paug_ref[2:3, :]
    y2p = paug_ref[3:4, :]
    area_p = paug_ref[4:5, :]
    px = paug_ref[5, :]
    py = paug_ref[6, :]
    inv_v0pw = paug_ref[7, :]
    inv_v0ph = paug_ref[8, :]
    inv_pw = paug_ref[9, :]
    inv_ph = paug_ref[10, :]

    t = targets_ref[0]
    tx1 = t[:, 0:1]
    ty1 = t[:, 1:2]
    tx2 = t[:, 2:3]
    ty2 = t[:, 3:4]
    area_t = (tx2 - tx1) * (ty2 - ty1)
    iw = jnp.maximum(jnp.minimum(x2p, tx2) - jnp.maximum(x1p, tx1), 0.0)
    ih = jnp.maximum(jnp.minimum(y2p, ty2) - jnp.maximum(y1p, ty1), 0.0)
    inter = iw * ih
    iou = inter / (area_p + area_t - inter)

    oio = lax.broadcasted_iota(jnp.int32, (_O, _P), 0)
    lio = lax.broadcasted_iota(jnp.int32, (_O, _P), 1)

    bv0 = jnp.max(iou, axis=0)
    bti0 = jnp.min(jnp.where(iou == bv0[None, :], oio, _O), axis=0)
    m = jnp.max(iou, axis=1, keepdims=True)
    bpi = jnp.min(jnp.where(iou == m, lio, _P), axis=1, keepdims=True)

    fm = lio == bpi
    f_o = jnp.max(jnp.where(fm, oio, -1), axis=0)
    f_any = f_o >= 0
    bv = jnp.where(f_any, 2.0, bv0)
    bti = jnp.where(f_any, f_o, bti0)

    onehotf = (oio == bti[None, :]).astype(jnp.float32)
    matched = lax.dot_general(
        targets_t_ref[0], onehotf, (((1,), (0,)), ((), ())),
        preferred_element_type=jnp.float32)
    mx1 = matched[0, :]
    my1 = matched[1, :]
    mx2 = matched[2, :]
    my2 = matched[3, :]
    labf = matched[4, :]

    conf_t = jnp.where(bv < _THRESHOLD, 0.0, labf + 1.0)
    pos = conf_t > 0.0
    posf = pos.astype(jnp.float32)

    gcx = ((mx1 + mx2) * 0.5 - px) * inv_v0pw
    gcy = ((my1 + my2) * 0.5 - py) * inv_v0ph
    gw = jnp.log((mx2 - mx1) * inv_pw) * (1.0 / _V1)
    gh = jnp.log((my2 - my1) * inv_ph) * (1.0 / _V1)
    ll = jnp.float32(0.0)
    for c, g in enumerate((gcx, gcy, gw, gh)):
        d = loc_ref[0, c, :] - g
        ad = jnp.abs(d)
        sl1 = jnp.where(ad < 1.0, 0.5 * d * d, ad - 0.5)
        ll = ll + jnp.sum(sl1 * posf)

    conf = conf_ref[0]
    rmax = jnp.max(conf, axis=0)
    s = jnp.sum(jnp.exp(conf - rmax[None, :]), axis=0)
    lse = jnp.log(s) + rmax
    cio = lax.broadcasted_iota(jnp.int32, (_NUM_CLASSES, _P), 0)
    conf_t_i = conf_t.astype(jnp.int32)
    g = jnp.sum(jnp.where(cio == conf_t_i[None, :], conf, 0.0), axis=0)
    ce = lse - g

    ce_m = jnp.where(pos, -2.0, ce)
    ce_ref[0, 0, :] = jnp.concatenate(
        [ce_m, jnp.full((_PPAD - _P,), -3.0, jnp.float32)])

    @pl.when(b == 0)
    def _init():
        ll_ref[0, 0] = 0.0
        lc_ref[0, 0] = 0.0

    ll_ref[0, 0] += ll
    lc_ref[0, 0] += jnp.sum(ce * posf)


def _sc_mine(ce_hbm, out_hbm, ce_v, row_v, red_v):
    img = lax.axis_index("s") * 2 + lax.axis_index("c")
    pltpu.sync_copy(ce_hbm.at[img], ce_v)

    zero16 = jnp.zeros((16,), jnp.float32)
    li = lax.iota(jnp.int32, 16)
    bfly = [li ^ 8, li ^ 4, li ^ 2, li ^ 1]

    def lane_red(v, op):
        for idx in bfly:
            red_v[...] = v
            v = op(v, plsc.load_gather(red_v, [idx]))
        return v

    def scan0(i, carry):
        mx, npv = carry
        for j in range(_U):
            ch = ce_v[pl.ds(i * (16 * _U) + 16 * j, 16)]
            mx = jnp.maximum(mx, ch)
            npv = npv + jnp.where(ch == -2.0, 1.0, 0.0)
        return mx, npv

    mx16, np16 = lax.fori_loop(
        0, _CHU, scan0, (jnp.full((16,), -10.0, jnp.float32), zero16))
    npos = lane_red(np16, jnp.add)
    k = jnp.minimum(_NEGPOS_RATIO * npos, jnp.float32(_P - 1))
    k = jnp.minimum(k, jnp.float32(_P) - npos)

    def bis(_, carry):
        lo, hi = carry
        mid = 0.5 * (lo + hi)

        def cnt_body(i, acc):
            for j in range(_U):
                ch = ce_v[pl.ds(i * (16 * _U) + 16 * j, 16)]
                acc = acc + jnp.where(ch > mid, 1.0, 0.0)
            return acc

        cn = lane_red(lax.fori_loop(0, _CHU, cnt_body, zero16), jnp.add)
        geq = cn >= k
        return jnp.where(geq, mid, lo), jnp.where(geq, hi, mid)

    lo, hi = lax.fori_loop(
        0, 24, bis,
        (jnp.full((16,), -1.0, jnp.float32),
         lane_red(mx16, jnp.maximum) + 1.0))

    def fin(i, carry):
        cnt, ssum = carry
        for j in range(_U):
            ch = ce_v[pl.ds(i * (16 * _U) + 16 * j, 16)]
            gt = ch > hi
            cnt = cnt + jnp.where(gt, 1.0, 0.0)
            ssum = ssum + jnp.where(gt, ch, 0.0)
        return cnt, ssum

    cnt16, ssum16 = lax.fori_loop(0, _CHU, fin, (zero16, zero16))
    s_top = lane_red(ssum16, jnp.add) + (k - lane_red(cnt16, jnp.add)) * hi

    row_v[...] = jnp.where(li == 0, s_top, jnp.where(li == 1, npos, 0.0))
    pltpu.sync_copy(row_v, out_hbm.at[img])


@jax.jit
def kernel(loc_data, conf_data, priors, targets):
    loc_t = jnp.transpose(loc_data, (0, 2, 1))
    conf_t2 = jnp.transpose(conf_data, (0, 2, 1))
    targets_t = jnp.transpose(targets, (0, 2, 1))

    px, py, pw, ph = priors[:, 0], priors[:, 1], priors[:, 2], priors[:, 3]
    paug = jnp.stack([
        px - pw * 0.5, py - ph * 0.5, px + pw * 0.5, py + ph * 0.5,
        pw * ph, px, py,
        1.0 / (_V0 * pw), 1.0 / (_V0 * ph), 1.0 / pw, 1.0 / ph,
    ], axis=0)

    scalar_spec = pl.BlockSpec((1, 1), lambda b: (0, 0),
                               memory_space=pltpu.SMEM)
    ll, lc_pos, ce_all = pl.pallas_call(
        _tc_body,
        grid=(_B,),
        in_specs=[
            pl.BlockSpec((1, 4, _P), lambda b: (b, 0, 0)),
            pl.BlockSpec((1, _NUM_CLASSES, _P), lambda b: (b, 0, 0)),
            pl.BlockSpec((11, _P), lambda b: (0, 0)),
            pl.BlockSpec((1, _O, 5), lambda b: (b, 0, 0)),
            pl.BlockSpec((1, 5, _O), lambda b: (b, 0, 0)),
        ],
        out_specs=[scalar_spec, scalar_spec,
                   pl.BlockSpec((1, 1, _PPAD), lambda b: (b, 0, 0))],
        out_shape=[jax.ShapeDtypeStruct((1, 1), jnp.float32),
                   jax.ShapeDtypeStruct((1, 1), jnp.float32),
                   jax.ShapeDtypeStruct((_B, 1, _PPAD), jnp.float32)],
    )(loc_t, conf_t2, paug, targets, targets_t)

    mine = pl.kernel(
        _sc_mine,
        out_type=jax.ShapeDtypeStruct((_B, 16), jnp.float32),
        mesh=plsc.VectorSubcoreMesh(core_axis_name="c",
                                    subcore_axis_name="s"),
        scratch_types=[pltpu.VMEM((_PPAD,), jnp.float32),
                       pltpu.VMEM((16,), jnp.float32),
                       pltpu.VMEM((16,), jnp.float32)],
        compiler_params=pltpu.CompilerParams(needs_layout_passes=False),
    )
    rows = mine(jnp.reshape(ce_all, (_B, _PPAD)))

    n = jnp.sum(rows[:, 1])
    lc = lc_pos[0, 0] + jnp.sum(rows[:, 0])
    return (ll[0, 0] / n, lc / n)
